# baseline (speedup 1.0000x reference)
import dataclasses
import functools

import jax
import jax.numpy as jnp
from jax import lax
from jax.experimental import pallas as pl
from jax.experimental.pallas import tpu as pltpu
from jax.experimental.pallas import tpu_sc as plsc

_N = 10000
_E = 320000
_D = 128
_R = 200
_NC = 2
_NS = 16
_NW = _NC * _NS
_CH = 32
_CPW0 = 512
_CPW1 = 128
_NCHUNK = _NS * (_CPW0 + _CPW1)
_IG = 8
_E_PAD = _NCHUNK * _CH
_NROWS = 10240
_RPT = _NROWS // _NS
_BN = 2000


def _idx_body(rel_ref, inv_ref, src_ref, ia_ref, ib_ref):
    inv = inv_ref[...]
    ia_ref[...] = inv * _R + rel_ref[...]
    ib_ref[...] = inv * _N + src_ref[...]


def _tab_body(x_ref, w_ref, btab_ref, self_ref):
    x = x_ref[...]
    for j in range(2):
        btab_ref[j] = jnp.dot(
            x, w_ref[:, j * _D:(j + 1) * _D], preferred_element_type=jnp.float32)
    self_ref[...] = jnp.dot(
        x, w_ref[:, 2 * _D:3 * _D], preferred_element_type=jnp.float32)


def _meta_body(m_ref, w_ref, b_ref, atab_ref, mnew_ref):
    m = m_ref[...]
    for j in range(2):
        atab_ref[j] = jnp.dot(
            m, w_ref[:, j * _D:(j + 1) * _D],
            preferred_element_type=jnp.float32) + b_ref[j]
    mnew_ref[...] = jnp.dot(
        m, w_ref[:, 2 * _D:3 * _D], preferred_element_type=jnp.float32) + b_ref[2]


def _comb_body(p_ref, c_ref, self_ref, b_ref, o_ref):
    s = p_ref[0] + p_ref[1]
    cnt = jnp.sum(c_ref[...], axis=1, keepdims=True)
    o_ref[...] = self_ref[...] + b_ref[...] + s / jnp.maximum(cnt, 1.0)


def _sc_edge_agg(a_tab, b_tab, ia, ib, dsts):
    mesh = plsc.VectorSubcoreMesh(core_axis_name="c", subcore_axis_name="s")
    cp = pltpu.CompilerParams()
    if "needs_layout_passes" in pltpu.CompilerParams.__dataclass_fields__:
        cp = dataclasses.replace(cp, needs_layout_passes=False)

    @functools.partial(
        pl.kernel,
        compiler_params=cp,
        out_type=(
            jax.ShapeDtypeStruct((_NC, _NS, _RPT, _D), jnp.float32),
            jax.ShapeDtypeStruct((_NC, _NS, _NROWS), jnp.float32),
        ),
        mesh=mesh,
        scratch_types=[
            pltpu.VMEM((_IG, _CH), jnp.int32),
            pltpu.VMEM((_IG, _CH), jnp.int32),
            pltpu.VMEM((_IG, _CH), jnp.int32),
        ] + [pltpu.VMEM((_CH, _D), jnp.float32)] * 8
          + [pltpu.VMEM((_NROWS,), jnp.float32),
             pltpu.VMEM_SHARED((_NROWS, _D), jnp.float32)]
          + [pltpu.SemaphoreType.DMA] * 16,
    )
    def edge_agg(a_hbm, b_hbm, ia_hbm, ib_hbm, dst_hbm, sums_out, cnt_out,
                 ia_v, ib_v, dst_v,
                 ra0, rb0, ra1, rb1, ra2, rb2, ra3, rb3,
                 cnt_v, acc_sh, *sems):
        c = lax.axis_index("c")
        s = lax.axis_index("s")
        base = s * _RPT
        cpw = jnp.where(c == 0, _CPW0, _CPW1)
        chunk0 = jnp.where(c == 0, s * _CPW0, _NS * _CPW0 + s * _CPW1)
        zeros16 = jnp.zeros((16,), jnp.float32)
        ones16 = jnp.full((16,), 1.0, jnp.float32)
        bufs = ((ra0, rb0), (ra1, rb1), (ra2, rb2), (ra3, rb3))
        rows = tuple(bufs[i] + tuple(sems[4 * i:4 * i + 4]) for i in range(4))

        def fire_gathers(j, slot):
            ra, rb, ga, gb = rows[slot][:4]
            pltpu.async_copy(a_hbm.at[ia_v.at[j]], ra, ga)
            pltpu.async_copy(b_hbm.at[ib_v.at[j]], rb, gb)

        def wait_gathers(j, slot):
            ra, rb, ga, gb = rows[slot][:4]
            pltpu.make_async_copy(a_hbm.at[ia_v.at[j]], ra, ga).wait()
            pltpu.make_async_copy(b_hbm.at[ib_v.at[j]], rb, gb).wait()

        def fire_scatters(j, slot):
            ra, rb = rows[slot][:2]
            sa, sb = rows[slot][4:]
            pltpu.async_copy(ra, acc_sh.at[dst_v.at[j]], sa, add=True)
            pltpu.async_copy(rb, acc_sh.at[dst_v.at[j]], sb, add=True)

        def wait_scatters(j, slot):
            ra, rb = rows[slot][:2]
            sa, sb = rows[slot][4:]
            pltpu.make_async_copy(ra, acc_sh.at[dst_v.at[j]], sa).wait()
            pltpu.make_async_copy(rb, acc_sh.at[dst_v.at[j]], sb).wait()

        def count(j):
            for k in range(_CH // 16):
                dstv = dst_v[j, pl.ds(k * 16, 16)]
                plsc.addupdate_scatter(cnt_v, [dstv], ones16)

        @pl.loop(0, _CH)
        def _zrow(r):
            @pl.loop(0, _D, step=16)
            def _zcol(k):
                ra0[r, pl.ds(k, 16)] = zeros16

        @pl.loop(0, _RPT, step=_CH)
        def _zacc(r0):
            pltpu.sync_copy(ra0, acc_sh.at[pl.ds(base + r0, _CH)])

        @pl.loop(0, _NROWS, step=16)
        def _zcnt(r0):
            cnt_v[pl.ds(r0, 16)] = zeros16

        plsc.subcore_barrier()

        @pl.loop(0, cpw, step=_IG)
        def _group(g):
            pltpu.sync_copy(ia_hbm.at[pl.ds(chunk0 + g, _IG)], ia_v)
            pltpu.sync_copy(ib_hbm.at[pl.ds(chunk0 + g, _IG)], ib_v)
            pltpu.sync_copy(dst_hbm.at[pl.ds(chunk0 + g, _IG)], dst_v)
            fire_gathers(0, 0)
            fire_gathers(1, 1)
            fire_gathers(2, 2)

            @pl.loop(0, _IG, step=4)
            def _quad(j):
                fire_gathers(j + 3, 3)
                for q in range(4):
                    count(j + q)
                for q in range(4):
                    wait_gathers(j + q, q)
                    fire_scatters(j + q, q)
                    if q >= 1:
                        wait_scatters(j + q - 1, q - 1)

                        @pl.when(j + q + 3 < _IG)
                        def _prefetch(q=q):
                            fire_gathers(j + q + 3, q - 1)

                wait_scatters(j + 3, 3)

        pltpu.sync_copy(cnt_v, cnt_out.at[c, s])
        plsc.subcore_barrier()
        pltpu.sync_copy(acc_sh.at[pl.ds(base, _RPT)], sums_out.at[c, s])

    return edge_agg(a_tab, b_tab, ia, ib, dsts)


def kernel(rel_emb, metarel_emb, edge_index, edge_rel, edge_inv,
           W_O_w, W_O_b, W_I_w, W_I_b, W_S_w, W_S_b, W_M_w, W_M_b):
    src = edge_index[0]
    dst = edge_index[1]
    pad = _E_PAD - _E
    nrow = _E_PAD // 128
    rel_p = jnp.pad(edge_rel, (0, pad)).reshape(nrow, 128)
    inv_p = jnp.pad(edge_inv, (0, pad)).reshape(nrow, 128)
    src_p = jnp.pad(src, (0, pad)).reshape(nrow, 128)
    dst_p = jnp.pad(dst, (0, pad), constant_values=_N).reshape(_NCHUNK, _CH)

    ia2, ib2 = pl.pallas_call(
        _idx_body,
        out_shape=[jax.ShapeDtypeStruct((nrow, 128), jnp.int32)] * 2,
    )(rel_p, inv_p, src_p)
    ia = ia2.reshape(_NCHUNK, _CH)
    ib = ib2.reshape(_NCHUNK, _CH)

    wb = jnp.concatenate(
        [W_I_w[:, _D:].T, W_O_w[:, _D:].T, W_S_w.T], axis=1)
    btab, self0 = pl.pallas_call(
        _tab_body,
        grid=(_N // _BN,),
        in_specs=[
            pl.BlockSpec((_BN, _D), lambda i: (i, 0)),
            pl.BlockSpec((_D, 3 * _D), lambda i: (0, 0)),
        ],
        out_specs=[
            pl.BlockSpec((2, _BN, _D), lambda i: (0, i, 0)),
            pl.BlockSpec((_BN, _D), lambda i: (i, 0)),
        ],
        out_shape=[
            jax.ShapeDtypeStruct((2, _N, _D), jnp.float32),
            jax.ShapeDtypeStruct((_N, _D), jnp.float32),
        ],
    )(rel_emb, wb)
    b_tab = btab.reshape(2 * _N, _D)

    wa = jnp.concatenate(
        [W_I_w[:, :_D].T, W_O_w[:, :_D].T, W_M_w.T], axis=1)
    ba = jnp.stack([W_I_b, W_O_b, W_M_b])[:, None, :]
    atab, metarel_new = pl.pallas_call(
        _meta_body,
        out_shape=[
            jax.ShapeDtypeStruct((2, _R, _D), jnp.float32),
            jax.ShapeDtypeStruct((_R, _D), jnp.float32),
        ],
    )(metarel_emb, wa, ba)
    a_tab = atab.reshape(2 * _R, _D)

    parts, cnts = _sc_edge_agg(a_tab, b_tab, ia, ib, dst_p)
    parts = parts.reshape(_NC, _NROWS, _D)
    cnts_t = cnts.reshape(_NW, _NROWS).T

    rel_emb_new = pl.pallas_call(
        _comb_body,
        grid=(_N // _BN,),
        in_specs=[
            pl.BlockSpec((_NC, _BN, _D), lambda i: (0, i, 0)),
            pl.BlockSpec((_BN, _NW), lambda i: (i, 0)),
            pl.BlockSpec((_BN, _D), lambda i: (i, 0)),
            pl.BlockSpec((1, _D), lambda i: (0, 0)),
        ],
        out_specs=pl.BlockSpec((_BN, _D), lambda i: (i, 0)),
        out_shape=jax.ShapeDtypeStruct((_N, _D), jnp.float32),
    )(parts, cnts_t, self0, W_S_b[None, :])

    return (rel_emb_new, metarel_new)


# speedup vs baseline: 1.0199x; 1.0199x over previous
import dataclasses
import functools

import jax
import jax.numpy as jnp
from jax import lax
from jax.experimental import pallas as pl
from jax.experimental.pallas import tpu as pltpu
from jax.experimental.pallas import tpu_sc as plsc

_N = 10000
_E = 320000
_D = 128
_R = 200
_NC = 2
_NS = 16
_NW = _NC * _NS
_CH = 64
_CPW0 = 256
_CPW1 = 64
_NCHUNK = _NS * (_CPW0 + _CPW1)
_IG = 8
_E_PAD = _NCHUNK * _CH
_NROWS = 10240
_RPT = _NROWS // _NS
_BN = 2000


def _idx_body(rel_ref, inv_ref, src_ref, ia_ref, ib_ref):
    inv = inv_ref[...]
    ia_ref[...] = inv * _R + rel_ref[...]
    ib_ref[...] = inv * _N + src_ref[...]


def _tab_body(x_ref, w_ref, btab_ref, self_ref):
    x = x_ref[...]
    for j in range(2):
        btab_ref[j] = jnp.dot(
            x, w_ref[:, j * _D:(j + 1) * _D], preferred_element_type=jnp.float32)
    self_ref[...] = jnp.dot(
        x, w_ref[:, 2 * _D:3 * _D], preferred_element_type=jnp.float32)


def _meta_body(m_ref, w_ref, b_ref, atab_ref, mnew_ref):
    m = m_ref[...]
    for j in range(2):
        atab_ref[j] = jnp.dot(
            m, w_ref[:, j * _D:(j + 1) * _D],
            preferred_element_type=jnp.float32) + b_ref[j]
    mnew_ref[...] = jnp.dot(
        m, w_ref[:, 2 * _D:3 * _D], preferred_element_type=jnp.float32) + b_ref[2]


def _comb_body(p_ref, c_ref, self_ref, b_ref, o_ref):
    s = p_ref[0] + p_ref[1]
    cnt = jnp.sum(c_ref[...], axis=1, keepdims=True)
    o_ref[...] = self_ref[...] + b_ref[...] + s / jnp.maximum(cnt, 1.0)


def _sc_edge_agg(a_tab, b_tab, ia, ib, dsts):
    mesh = plsc.VectorSubcoreMesh(core_axis_name="c", subcore_axis_name="s")
    cp = pltpu.CompilerParams()
    if "needs_layout_passes" in pltpu.CompilerParams.__dataclass_fields__:
        cp = dataclasses.replace(cp, needs_layout_passes=False)

    @functools.partial(
        pl.kernel,
        compiler_params=cp,
        out_type=(
            jax.ShapeDtypeStruct((_NC, _NS, _RPT, _D), jnp.float32),
            jax.ShapeDtypeStruct((_NC, _NS, _NROWS), jnp.float32),
        ),
        mesh=mesh,
        scratch_types=[
            pltpu.VMEM((_IG, _CH), jnp.int32),
            pltpu.VMEM((_IG, _CH), jnp.int32),
            pltpu.VMEM((_IG, _CH), jnp.int32),
        ] + [pltpu.VMEM((_CH, _D), jnp.float32)] * 4
          + [pltpu.VMEM((_NROWS,), jnp.float32),
             pltpu.VMEM_SHARED((_NROWS, _D), jnp.float32)]
          + [pltpu.SemaphoreType.DMA] * 8,
    )
    def edge_agg(a_hbm, b_hbm, ia_hbm, ib_hbm, dst_hbm, sums_out, cnt_out,
                 ia_v, ib_v, dst_v,
                 ra0, rb0, ra1, rb1,
                 cnt_v, acc_sh, *sems):
        c = lax.axis_index("c")
        s = lax.axis_index("s")
        base = s * _RPT
        cpw = jnp.where(c == 0, _CPW0, _CPW1)
        chunk0 = jnp.where(c == 0, s * _CPW0, _NS * _CPW0 + s * _CPW1)
        zeros16 = jnp.zeros((16,), jnp.float32)
        ones16 = jnp.full((16,), 1.0, jnp.float32)
        bufs = ((ra0, rb0), (ra1, rb1))
        rows = tuple(bufs[i] + tuple(sems[4 * i:4 * i + 4]) for i in range(2))

        def fire_gathers(j, slot):
            ra, rb, ga, gb = rows[slot][:4]
            pltpu.async_copy(a_hbm.at[ia_v.at[j]], ra, ga)
            pltpu.async_copy(b_hbm.at[ib_v.at[j]], rb, gb)

        def wait_gathers(j, slot):
            ra, rb, ga, gb = rows[slot][:4]
            pltpu.make_async_copy(a_hbm.at[ia_v.at[j]], ra, ga).wait()
            pltpu.make_async_copy(b_hbm.at[ib_v.at[j]], rb, gb).wait()

        def fire_scatters(j, slot):
            ra, rb = rows[slot][:2]
            sa, sb = rows[slot][4:]
            pltpu.async_copy(ra, acc_sh.at[dst_v.at[j]], sa, add=True)
            pltpu.async_copy(rb, acc_sh.at[dst_v.at[j]], sb, add=True)

        def wait_scatters(j, slot):
            ra, rb = rows[slot][:2]
            sa, sb = rows[slot][4:]
            pltpu.make_async_copy(ra, acc_sh.at[dst_v.at[j]], sa).wait()
            pltpu.make_async_copy(rb, acc_sh.at[dst_v.at[j]], sb).wait()

        def count(j):
            for k in range(_CH // 16):
                dstv = dst_v[j, pl.ds(k * 16, 16)]
                plsc.addupdate_scatter(cnt_v, [dstv], ones16)

        @pl.loop(0, _CH)
        def _zrow(r):
            @pl.loop(0, _D, step=16)
            def _zcol(k):
                ra0[r, pl.ds(k, 16)] = zeros16

        @pl.loop(0, _RPT, step=_CH)
        def _zacc(r0):
            pltpu.sync_copy(ra0, acc_sh.at[pl.ds(base + r0, _CH)])

        @pl.loop(0, _NROWS, step=16)
        def _zcnt(r0):
            cnt_v[pl.ds(r0, 16)] = zeros16

        plsc.subcore_barrier()

        @pl.loop(0, cpw, step=_IG)
        def _group(g):
            pltpu.sync_copy(ia_hbm.at[pl.ds(chunk0 + g, _IG)], ia_v)
            pltpu.sync_copy(ib_hbm.at[pl.ds(chunk0 + g, _IG)], ib_v)
            pltpu.sync_copy(dst_hbm.at[pl.ds(chunk0 + g, _IG)], dst_v)
            fire_gathers(0, 0)

            @pl.loop(0, _IG, step=2)
            def _pair(j):
                fire_gathers(j + 1, 1)
                count(j)
                count(j + 1)
                wait_gathers(j, 0)
                fire_scatters(j, 0)
                wait_gathers(j + 1, 1)
                fire_scatters(j + 1, 1)
                wait_scatters(j, 0)

                @pl.when(j + 2 < _IG)
                def _prefetch():
                    fire_gathers(j + 2, 0)

                wait_scatters(j + 1, 1)

        pltpu.sync_copy(cnt_v, cnt_out.at[c, s])
        plsc.subcore_barrier()
        pltpu.sync_copy(acc_sh.at[pl.ds(base, _RPT)], sums_out.at[c, s])

    return edge_agg(a_tab, b_tab, ia, ib, dsts)


def kernel(rel_emb, metarel_emb, edge_index, edge_rel, edge_inv,
           W_O_w, W_O_b, W_I_w, W_I_b, W_S_w, W_S_b, W_M_w, W_M_b):
    src = edge_index[0]
    dst = edge_index[1]
    pad = _E_PAD - _E
    nrow = _E_PAD // 128
    rel_p = jnp.pad(edge_rel, (0, pad)).reshape(nrow, 128)
    inv_p = jnp.pad(edge_inv, (0, pad)).reshape(nrow, 128)
    src_p = jnp.pad(src, (0, pad)).reshape(nrow, 128)
    dst_p = jnp.pad(dst, (0, pad), constant_values=_N).reshape(_NCHUNK, _CH)

    ia2, ib2 = pl.pallas_call(
        _idx_body,
        out_shape=[jax.ShapeDtypeStruct((nrow, 128), jnp.int32)] * 2,
    )(rel_p, inv_p, src_p)
    ia = ia2.reshape(_NCHUNK, _CH)
    ib = ib2.reshape(_NCHUNK, _CH)

    wb = jnp.concatenate(
        [W_I_w[:, _D:].T, W_O_w[:, _D:].T, W_S_w.T], axis=1)
    btab, self0 = pl.pallas_call(
        _tab_body,
        grid=(_N // _BN,),
        in_specs=[
            pl.BlockSpec((_BN, _D), lambda i: (i, 0)),
            pl.BlockSpec((_D, 3 * _D), lambda i: (0, 0)),
        ],
        out_specs=[
            pl.BlockSpec((2, _BN, _D), lambda i: (0, i, 0)),
            pl.BlockSpec((_BN, _D), lambda i: (i, 0)),
        ],
        out_shape=[
            jax.ShapeDtypeStruct((2, _N, _D), jnp.float32),
            jax.ShapeDtypeStruct((_N, _D), jnp.float32),
        ],
    )(rel_emb, wb)
    b_tab = btab.reshape(2 * _N, _D)

    wa = jnp.concatenate(
        [W_I_w[:, :_D].T, W_O_w[:, :_D].T, W_M_w.T], axis=1)
    ba = jnp.stack([W_I_b, W_O_b, W_M_b])[:, None, :]
    atab, metarel_new = pl.pallas_call(
        _meta_body,
        out_shape=[
            jax.ShapeDtypeStruct((2, _R, _D), jnp.float32),
            jax.ShapeDtypeStruct((_R, _D), jnp.float32),
        ],
    )(metarel_emb, wa, ba)
    a_tab = atab.reshape(2 * _R, _D)

    parts, cnts = _sc_edge_agg(a_tab, b_tab, ia, ib, dst_p)
    parts = parts.reshape(_NC, _NROWS, _D)
    cnts_t = cnts.reshape(_NW, _NROWS).T

    rel_emb_new = pl.pallas_call(
        _comb_body,
        grid=(_N // _BN,),
        in_specs=[
            pl.BlockSpec((_NC, _BN, _D), lambda i: (0, i, 0)),
            pl.BlockSpec((_BN, _NW), lambda i: (i, 0)),
            pl.BlockSpec((_BN, _D), lambda i: (i, 0)),
            pl.BlockSpec((1, _D), lambda i: (0, 0)),
        ],
        out_specs=pl.BlockSpec((_BN, _D), lambda i: (i, 0)),
        out_shape=jax.ShapeDtypeStruct((_N, _D), jnp.float32),
    )(parts, cnts_t, self0, W_S_b[None, :])

    return (rel_emb_new, metarel_new)
